# baseline (speedup 1.0000x reference)
import functools

import jax
import jax.numpy as jnp
from jax import lax
from jax.experimental import pallas as pl
from jax.experimental.pallas import tpu as pltpu
from jax.experimental.pallas import tpu_sc as plsc

B = 16384
D = 3072
NUM_CLASSES = 10
NC, NS, L = 2, 16, 16
NW = NC * NS
ROWS_W = B // NW
FLAT_W = ROWS_W * NUM_CLASSES
GROUPS = ROWS_W // L

_mesh = plsc.VectorSubcoreMesh(core_axis_name="c", subcore_axis_name="s")


@functools.partial(
    pl.kernel,
    out_type=jax.ShapeDtypeStruct((B * NUM_CLASSES,), jnp.float32),
    mesh=_mesh,
    scratch_types=[
        pltpu.VMEM((ROWS_W,), jnp.int32),
        pltpu.VMEM((FLAT_W,), jnp.float32),
    ],
    compiler_params=pltpu.CompilerParams(needs_layout_passes=False),
)
def _sc_one_hot(y_hbm, out_hbm, idx_v, buf):
    wid = lax.axis_index("s") * NC + lax.axis_index("c")
    pltpu.sync_copy(y_hbm.at[pl.ds(wid * ROWS_W, ROWS_W)], idx_v)

    zeros = jnp.zeros((L,), jnp.float32)
    for k in range(FLAT_W // L):
        buf[pl.ds(k * L, L)] = zeros

    ones = jnp.ones((L,), jnp.float32)
    lane = lax.broadcasted_iota(jnp.int32, (L,), 0)
    for j in range(GROUPS):
        yv = idx_v[pl.ds(j * L, L)]
        flat = (lane + j * L) * NUM_CLASSES + yv
        plsc.store_scatter(buf, [flat], ones)

    pltpu.sync_copy(buf, out_hbm.at[pl.ds(wid * FLAT_W, FLAT_W)])


def kernel(x, y):
    oh_flat = _sc_one_hot(y)
    return (x, oh_flat.reshape(B, NUM_CLASSES))


# speedup vs baseline: 1.1539x; 1.1539x over previous
import jax
import jax.numpy as jnp
from jax.experimental import pallas as pl
from jax.experimental.pallas import tpu as pltpu

B = 16384
D = 3072
NUM_CLASSES = 10
RB = 512
NBLK = B // RB


def _body(x_ref, y_ref, xout_ref, oh_ref):
    xout_ref[...] = x_ref[...]
    yv = y_ref[...]
    iota = jax.lax.broadcasted_iota(jnp.int32, (RB, NUM_CLASSES), 1)
    oh_ref[...] = (yv == iota).astype(jnp.float32)


def kernel(x, y):
    y2 = y.reshape(B, 1)
    x_out, one_hot = pl.pallas_call(
        _body,
        grid=(NBLK,),
        in_specs=[
            pl.BlockSpec((RB, D), lambda i: (i, 0)),
            pl.BlockSpec((RB, 1), lambda i: (i, 0)),
        ],
        out_specs=[
            pl.BlockSpec((RB, D), lambda i: (i, 0)),
            pl.BlockSpec((RB, NUM_CLASSES), lambda i: (i, 0)),
        ],
        out_shape=[
            jax.ShapeDtypeStruct((B, D), jnp.float32),
            jax.ShapeDtypeStruct((B, NUM_CLASSES), jnp.float32),
        ],
        compiler_params=pltpu.CompilerParams(
            dimension_semantics=("arbitrary",),
        ),
    )(x, y2)
    return (x_out, one_hot)
